# Initial kernel scaffold; baseline (speedup 1.0000x reference)
import jax
import jax.numpy as jnp
from jax.experimental import pallas as pl


def kernel(word, pos1, pos2, word_table, pos1_table, pos2_table):
    raise NotImplementedError("write your pallas kernel here")


# speedup vs baseline: 5.1814x; 5.1814x over previous
import functools

import jax
import jax.numpy as jnp
from jax import lax
from jax.experimental import pallas as pl
from jax.experimental.pallas import tpu as pltpu
from jax.experimental.pallas import tpu_sc as plsc

B = 4096
L = 200
N = B * L
EMB = 64
PD = 32
NW = 32
G = 128
NG = N // (NW * G)

_mesh = plsc.VectorSubcoreMesh(core_axis_name="c", subcore_axis_name="s")


@functools.partial(
    pl.kernel,
    mesh=_mesh,
    compiler_params=pltpu.CompilerParams(use_tc_tiling_on_sc=False),
    out_type=jax.ShapeDtypeStruct((N, EMB + 2 * PD), jnp.float32),
    scratch_types=[
        pltpu.VMEM((NG, G), jnp.int32),
        pltpu.VMEM((NG, G), jnp.int32),
        pltpu.VMEM((NG, G), jnp.int32),
        pltpu.VMEM((G, EMB), jnp.float32),
        pltpu.VMEM((G, PD), jnp.float32),
        pltpu.VMEM((G, PD), jnp.float32),
        pltpu.SemaphoreType.DMA,
        pltpu.SemaphoreType.DMA,
        pltpu.SemaphoreType.DMA,
    ],
)
def _sc_embed(word_hbm, pos1_hbm, pos2_hbm, wtab_hbm, p1tab_hbm, p2tab_hbm,
              out_hbm, widx_v, p1idx_v, p2idx_v, wrows_v, p1rows_v, p2rows_v,
              wsem, p1sem, p2sem):
    wid = lax.axis_index("s") * 2 + lax.axis_index("c")
    gbase = wid * NG

    pltpu.sync_copy(word_hbm.at[pl.ds(gbase, NG)], widx_v)
    pltpu.sync_copy(pos1_hbm.at[pl.ds(gbase, NG)], p1idx_v)
    pltpu.sync_copy(pos2_hbm.at[pl.ds(gbase, NG)], p2idx_v)

    def body(g, carry):
        row0 = (gbase + g) * G
        pltpu.async_copy(wtab_hbm.at[widx_v.at[g]], wrows_v, wsem).wait()
        pltpu.sync_copy(wrows_v, out_hbm.at[pl.ds(row0, G), pl.ds(0, EMB)])
        pltpu.async_copy(p1tab_hbm.at[p1idx_v.at[g]], p1rows_v, p1sem).wait()
        pltpu.sync_copy(p1rows_v, out_hbm.at[pl.ds(row0, G), pl.ds(EMB, PD)])
        pltpu.async_copy(p2tab_hbm.at[p2idx_v.at[g]], p2rows_v, p2sem).wait()
        pltpu.sync_copy(p2rows_v,
                        out_hbm.at[pl.ds(row0, G), pl.ds(EMB + PD, PD)])
        return carry

    lax.fori_loop(0, NG, body, 0)


def kernel(word, pos1, pos2, word_table, pos1_table, pos2_table):
    word2d = jnp.reshape(word, (N // G, G))
    pos1_2d = jnp.reshape(pos1, (N // G, G))
    pos2_2d = jnp.reshape(pos2, (N // G, G))
    out = _sc_embed(word2d, pos1_2d, pos2_2d,
                    word_table, pos1_table, pos2_table)
    return jnp.reshape(out, (B, L, EMB + 2 * PD))


# speedup vs baseline: 5.8807x; 1.1350x over previous
import functools

import jax
import jax.numpy as jnp
from jax import lax
from jax.experimental import pallas as pl
from jax.experimental.pallas import tpu as pltpu
from jax.experimental.pallas import tpu_sc as plsc

B = 4096
L = 200
N = B * L
EMB = 64
PD = 32
OUT_D = EMB + 2 * PD
NW = 32
G = 128
NG = N // (NW * G)
NSEG = 2
SEGG = NG // NSEG
NBUF = 4

_mesh = plsc.VectorSubcoreMesh(core_axis_name="c", subcore_axis_name="s")


@functools.partial(
    pl.kernel,
    mesh=_mesh,
    compiler_params=pltpu.CompilerParams(use_tc_tiling_on_sc=False),
    out_type=jax.ShapeDtypeStruct((N, OUT_D), jnp.float32),
    scratch_types=[
        pltpu.VMEM((SEGG, G), jnp.int32),
        pltpu.VMEM((SEGG, G), jnp.int32),
        pltpu.VMEM((SEGG, G), jnp.int32),
        pltpu.VMEM((NBUF, G, EMB), jnp.float32),
        pltpu.VMEM((NBUF, G, PD), jnp.float32),
        pltpu.VMEM((NBUF, G, PD), jnp.float32),
        pltpu.SemaphoreType.DMA,
        pltpu.SemaphoreType.DMA,
        pltpu.SemaphoreType.DMA,
        pltpu.SemaphoreType.DMA,
        pltpu.SemaphoreType.DMA,
        pltpu.SemaphoreType.DMA,
        pltpu.SemaphoreType.DMA,
        pltpu.SemaphoreType.DMA,
    ],
)
def _sc_embed(word_hbm, pos1_hbm, pos2_hbm, wtab_hbm, p1tab_hbm, p2tab_hbm,
              out_hbm, widx_v, p1idx_v, p2idx_v, wrows_v, p1rows_v, p2rows_v,
              gs0, gs1, gs2, gs3, ws0, ws1, ws2, ws3):
    gsems = [gs0, gs1, gs2, gs3]
    wsems = [ws0, ws1, ws2, ws3]
    wid = lax.axis_index("s") * 2 + lax.axis_index("c")
    gbase = wid * NG

    def gather_start(g, b):
        pltpu.async_copy(wtab_hbm.at[widx_v.at[g]], wrows_v.at[b], gsems[b])
        pltpu.async_copy(p1tab_hbm.at[p1idx_v.at[g]], p1rows_v.at[b], gsems[b])
        pltpu.async_copy(p2tab_hbm.at[p2idx_v.at[g]], p2rows_v.at[b], gsems[b])

    def gather_wait(b):
        pltpu.make_async_copy(wtab_hbm.at[pl.ds(0, G)], wrows_v.at[b],
                              gsems[b]).wait()
        pltpu.make_async_copy(p1tab_hbm.at[pl.ds(0, G)], p1rows_v.at[b],
                              gsems[b]).wait()
        pltpu.make_async_copy(p2tab_hbm.at[pl.ds(0, G)], p2rows_v.at[b],
                              gsems[b]).wait()

    def write_start(g, b, seg_off):
        row0 = (gbase + seg_off + g) * G
        pltpu.async_copy(wrows_v.at[b],
                         out_hbm.at[pl.ds(row0, G), pl.ds(0, EMB)], wsems[b])
        pltpu.async_copy(p1rows_v.at[b],
                         out_hbm.at[pl.ds(row0, G), pl.ds(EMB, PD)], wsems[b])
        pltpu.async_copy(p2rows_v.at[b],
                         out_hbm.at[pl.ds(row0, G), pl.ds(EMB + PD, PD)],
                         wsems[b])

    def write_wait(b):
        row0 = gbase * G
        pltpu.make_async_copy(wrows_v.at[b],
                              out_hbm.at[pl.ds(row0, G), pl.ds(0, EMB)],
                              wsems[b]).wait()
        pltpu.make_async_copy(p1rows_v.at[b],
                              out_hbm.at[pl.ds(row0, G), pl.ds(EMB, PD)],
                              wsems[b]).wait()
        pltpu.make_async_copy(p2rows_v.at[b],
                              out_hbm.at[pl.ds(row0, G), pl.ds(EMB + PD, PD)],
                              wsems[b]).wait()

    for seg in range(NSEG):
        seg_off = seg * SEGG
        pltpu.sync_copy(word_hbm.at[pl.ds(gbase + seg_off, SEGG)], widx_v)
        pltpu.sync_copy(pos1_hbm.at[pl.ds(gbase + seg_off, SEGG)], p1idx_v)
        pltpu.sync_copy(pos2_hbm.at[pl.ds(gbase + seg_off, SEGG)], p2idx_v)

        gather_start(0, 0)
        gather_start(1, 1)

        def outer(i, carry):
            go = i * NBUF
            for b in range(NBUF):
                g = go + b
                gather_wait(b)
                write_start(g, b, seg_off)
                bn = (b + 2) % NBUF

                @pl.when(g >= 2)
                def _():
                    write_wait(bn)

                @pl.when(g + 2 < SEGG)
                def _():
                    gather_start(g + 2, bn)
            return carry

        lax.fori_loop(0, SEGG // NBUF, outer, 0)
        write_wait((SEGG - 2) % NBUF)
        write_wait((SEGG - 1) % NBUF)


def kernel(word, pos1, pos2, word_table, pos1_table, pos2_table):
    word2d = jnp.reshape(word, (N // G, G))
    pos1_2d = jnp.reshape(pos1, (N // G, G))
    pos2_2d = jnp.reshape(pos2, (N // G, G))
    out = _sc_embed(word2d, pos1_2d, pos2_2d,
                    word_table, pos1_table, pos2_table)
    return jnp.reshape(out, (B, L, OUT_D))
